# Initial kernel scaffold; baseline (speedup 1.0000x reference)
import jax
import jax.numpy as jnp
from jax.experimental import pallas as pl


def kernel(x, edge_index, edge_attr, shift, W1, a1_src, a1_dst, We1, a1_edge, b1, W2, a2_src, a2_dst, We2, a2_edge, b2, W_l2, b_l2, W_l3, b_l3, Wm1, bm1, Wm2, bm2, W_l4, b_l4):
    raise NotImplementedError("write your pallas kernel here")


# speedup vs baseline: 1.4946x; 1.4946x over previous
import functools

import jax
import jax.numpy as jnp
from jax.experimental import pallas as pl
from jax.experimental.pallas import tpu as pltpu

N = 10000
E = 160000
D = 128
H = 128
HP = 144
OUT = 313

_BM = 640


def _edge_mlp_body(u0_ref, wm1_ref, bm1_ref, wm2_ref, bm2_ref, wl4_ref, bl4_ref,
                   out_ref):
    u0 = jnp.maximum(u0_ref[...], 0.0)
    u1 = jnp.dot(u0, wm1_ref[...], preferred_element_type=jnp.float32)
    u1 = jnp.maximum(u1 + bm1_ref[...], 0.0)
    u2 = jnp.dot(u1, wm2_ref[...], preferred_element_type=jnp.float32)
    u2 = jnp.maximum(u2 + bm2_ref[...], 0.0)
    yb = jnp.dot(u2, wl4_ref[...], preferred_element_type=jnp.float32)
    out_ref[...] = yb + bl4_ref[...]


def _edge_mlp(u0, wm1p, bm1p, wm2p, bm2p, wl4p, bl4p):
    grid = (E // _BM,)
    return pl.pallas_call(
        _edge_mlp_body,
        grid=grid,
        in_specs=[
            pl.BlockSpec((_BM, HP), lambda i: (i, 0)),
            pl.BlockSpec((HP, HP), lambda i: (0, 0)),
            pl.BlockSpec((1, HP), lambda i: (0, 0)),
            pl.BlockSpec((HP, HP), lambda i: (0, 0)),
            pl.BlockSpec((1, HP), lambda i: (0, 0)),
            pl.BlockSpec((HP, OUT), lambda i: (0, 0)),
            pl.BlockSpec((1, OUT), lambda i: (0, 0)),
        ],
        out_specs=pl.BlockSpec((_BM, OUT), lambda i: (i, 0)),
        out_shape=jax.ShapeDtypeStruct((E, OUT), jnp.float32),
    )(u0, wm1p, bm1p, wm2p, bm2p, wl4p, bl4p)


def _pad2(a, r, c):
    return jnp.pad(a, ((0, r - a.shape[0]), (0, c - a.shape[1])))


def kernel(x, edge_index, edge_attr, shift, W1, a1_src, a1_dst, We1, a1_edge, b1,
           W2, a2_src, a2_dst, We2, a2_edge, b2, W_l2, b_l2, W_l3, b_l3,
           Wm1, bm1, Wm2, bm2, W_l4, b_l4):
    src = edge_index[0]
    dst = edge_index[1]
    mask = src != dst
    maskf = mask.astype(jnp.float32)
    cnt = jax.ops.segment_sum(maskf, dst, num_segments=N)
    loop_attr = jax.ops.segment_sum(edge_attr * maskf[:, None], dst,
                                    num_segments=N) / jnp.maximum(cnt, 1.0)[:, None]

    def gat(xin, W, a_s, a_d, We, a_e, b):
        h = xin @ W
        asn = h @ a_s
        adn = h @ a_d
        c = We @ a_e
        ae = edge_attr @ c
        ae_loop = loop_attr @ c
        alpha = asn[src] + adn[dst] + ae
        alpha = jnp.where(alpha >= 0, alpha, 0.2 * alpha)
        ex = jnp.where(mask, jnp.exp(alpha), 0.0)
        al = asn + adn + ae_loop
        al = jnp.where(al >= 0, al, 0.2 * al)
        exl = jnp.exp(al)
        num = jax.ops.segment_sum(h[src] * ex[:, None], dst, num_segments=N)
        num = num + h * exl[:, None]
        den = jax.ops.segment_sum(ex, dst, num_segments=N) + exl
        return num / (den[:, None] + 1e-16) + b

    y0 = jax.nn.relu(gat(x, W1, a1_src, a1_dst, We1, a1_edge, b1))
    y1 = jax.nn.relu(gat(y0, W2, a2_src, a2_dst, We2, a2_edge, b2))
    y4 = jax.nn.relu((y0 + y1) @ W_l2 + b_l2)

    z = y4 @ W_l3[:H] + 0.5 * b_l3
    zp = jnp.pad(z, ((0, 0), (0, HP - (H + 2))))
    eb = jnp.pad(edge_attr @ W_l3[H:], ((0, 0), (0, HP - (H + 2))))
    u0 = zp[src] + zp[dst] + eb

    wm1p = _pad2(Wm1, HP, HP)
    wm2p = _pad2(Wm2, HP, HP)
    wl4p = jnp.pad(W_l4, ((0, HP - (H + 2)), (0, 0)))
    bm1p = jnp.pad(bm1, (0, HP - (H + 2)))[None, :]
    bm2p = jnp.pad(bm2, (0, HP - (H + 2)))[None, :]
    bl4p = b_l4[None, :]

    return _edge_mlp(u0, wm1p, bm1p, wm2p, bm2p, wl4p, bl4p)
